# baseline (speedup 1.0000x reference)
import functools

import jax
import jax.numpy as jnp
from jax import lax
from jax.experimental import pallas as pl
from jax.experimental.pallas import tpu as pltpu
from jax.experimental.pallas import tpu_sc as plsc

D = 64
SCALE = 8.0
VOCAB = 1_000_000
NG = VOCAB // 8
NB = 4096
NSQ = 200
NC = 2
NW = 32

RW = 25
G_PER_W = NG // RW
RCH = 10
RN = G_PER_W // RCH

CHUNK = 128
NCHUNK = NB // 128 * NSQ // NW
NBUF = 4


def _make_linearize():
    mesh = plsc.VectorSubcoreMesh(core_axis_name="c", subcore_axis_name="s")

    @functools.partial(
        pl.kernel,
        mesh=mesh,
        out_type=jax.ShapeDtypeStruct((VOCAB // 2, 128), jnp.float32),
        compiler_params=pltpu.CompilerParams(use_tc_tiling_on_sc=True,
                                             needs_layout_passes=False),
        scratch_types=(
            [pltpu.VMEM((4, RCH, 8, D), jnp.float32),
             pltpu.VMEM((4, RCH * 4, 128), jnp.float32)]
            + [pltpu.SemaphoreType.DMA] * 8
        ),
    )
    def linearize(table_hbm, out_hbm, a_v, b_v, *sems):
        rsem = sems[:4]
        wsem = sems[4:]
        wid = lax.axis_index("s") * NC + lax.axis_index("c")
        base = wid * G_PER_W

        def rd(g, sl):
            pltpu.async_copy(
                table_hbm.at[pl.ds(base + g * RCH, RCH)], a_v.at[sl],
                rsem[sl])

        def rd_wait(sl):
            pltpu.make_async_copy(
                table_hbm.at[pl.ds(base, RCH)], a_v.at[sl], rsem[sl]).wait()

        def wr(g, sl):
            pltpu.async_copy(
                b_v.at[sl],
                out_hbm.at[pl.ds((base + g * RCH) * 4, RCH * 4)], wsem[sl])

        def wr_wait(sl):
            pltpu.make_async_copy(
                b_v.at[sl], out_hbm.at[pl.ds(0, RCH * 4)], wsem[sl]).wait()

        def repack(sl):
            def body(gq, carry):
                for h in range(4):
                    q = gq * 4 + h
                    for half in range(2):
                        s = 2 * h + half
                        for j0 in range(0, D, 16):
                            v = a_v[sl, gq, s, pl.ds(j0, 16)]
                            b_v[sl, q, pl.ds(half * D + j0, 16)] = v
                return carry
            lax.fori_loop(0, RCH, body, 0)

        @pl.when(wid < RW)
        def _():
            for sl in range(4):
                rd(sl, sl)
            def outer(i, carry):
                for k in range(4):
                    g = 4 * i + k
                    rd_wait(k)

                    @pl.when(g >= 4)
                    def _w():
                        wr_wait(k)
                    repack(k)
                    wr(g, k)

                    @pl.when(g + 4 < RN)
                    def _r():
                        rd(g + 4, k)
                return carry
            lax.fori_loop(0, RN // 4, outer, 0)
            for sl in range(4):
                wr_wait(sl)

    return linearize


def _make_gather():
    mesh = plsc.VectorSubcoreMesh(core_axis_name="c", subcore_axis_name="s")

    @functools.partial(
        pl.kernel,
        mesh=mesh,
        out_type=jax.ShapeDtypeStruct((NSQ, 8, NB // 128, 8, 128),
                                      jnp.float32),
        compiler_params=pltpu.CompilerParams(use_tc_tiling_on_sc=False,
                                             needs_layout_passes=False),
        scratch_types=(
            [pltpu.VMEM((2, NB), jnp.int32),
             pltpu.VMEM((NBUF, CHUNK, D), jnp.float32),
             pltpu.VMEM((NBUF, 8, 8, 128), jnp.float32)]
            + [pltpu.SemaphoreType.DMA] * (1 + 2 * NBUF)
        ),
    )
    def gather(xt_hbm, tab_hbm, out_hbm, idx_v, g_v, o_v, *sems):
        isem = sems[0]
        gsem = sems[1:1 + NBUF]
        ssem = sems[1 + NBUF:]
        wid = lax.axis_index("s") * NC + lax.axis_index("c")
        t0 = wid * NCHUNK

        def chunk_pos(g):
            t = t0 + g
            return t // 32, t % 32

        def idx_load(s):
            pltpu.async_copy(xt_hbm.at[s], idx_v.at[s % 2], isem)

        def idx_wait():
            pltpu.make_async_copy(
                xt_hbm.at[0], idx_v.at[0], isem).wait()

        def issue_gather(g, b):
            s, bh = chunk_pos(g)
            row = s % 2
            pltpu.async_copy(
                tab_hbm.at[idx_v.at[row, pl.ds(bh * CHUNK, CHUNK)]],
                g_v.at[b], gsem[b])

        def wait_gather(b):
            pltpu.make_async_copy(
                tab_hbm.at[idx_v.at[0, pl.ds(0, CHUNK)]], g_v.at[b],
                gsem[b]).wait()

        def issue_out(g, b):
            s, bh = chunk_pos(g)
            pltpu.async_copy(o_v.at[b], out_hbm.at[s, :, bh], ssem[b])

        def wait_out(b):
            pltpu.make_async_copy(
                o_v.at[b], out_hbm.at[0, :, 0], ssem[b]).wait()

        lane = lax.iota(jnp.int32, 16)

        def transpose_scale(b):
            for jh in range(8):
                for jl in range(8):
                    cols = jnp.zeros((16,), jnp.int32) + (jh * 8 + jl)
                    for lg in range(8):
                        v = plsc.load_gather(g_v.at[b],
                                             [lane + lg * 16, cols])
                        o_v[b, jh, jl, pl.ds(lg * 16, 16)] = v * SCALE

        s_first, _ = chunk_pos(0)
        idx_load(s_first)
        idx_wait()

        @pl.when(t0 % 32 != 0)
        def _():
            idx_load(s_first + 1)
        for b in range(NBUF):
            issue_gather(b, b)

        def outer(i, carry):
            g0 = i * NBUF
            for b in range(NBUF):
                g = g0 + b
                wait_gather(b)

                @pl.when(g >= NBUF)
                def _():
                    wait_out(b)

                t_pre = t0 + g + NBUF

                @pl.when(jnp.logical_and(t_pre % 32 == 0,
                                         g + NBUF < NCHUNK))
                def _():
                    idx_wait()

                @pl.when(jnp.logical_and(t_pre % 32 == NBUF,
                                         g + NBUF < NCHUNK))
                def _():
                    s_next = t_pre // 32 + 1
                    need = jnp.logical_and(
                        s_next * 32 < t0 + NCHUNK, s_next < NSQ)

                    @pl.when(need)
                    def _():
                        idx_load(s_next)
                transpose_scale(b)
                issue_out(g, b)

                @pl.when(g + NBUF < NCHUNK)
                def _():
                    issue_gather(g + NBUF, b)
            return carry
        lax.fori_loop(0, NCHUNK // NBUF, outer, 0)

        for b in range(NBUF):
            wait_out(b)

    return gather


_linearize = _make_linearize()
_gather = _make_gather()


def kernel(x, table):
    tab_lin = _linearize(table.reshape(NG, 8, D))
    out5 = _gather(x.T.astype(jnp.int32),
                   tab_lin.reshape(VOCAB, D))
    t = out5.transpose(2, 4, 0, 1, 3)
    return t.reshape(NB, NSQ, D)


# speedup vs baseline: 1.9363x; 1.9363x over previous
import functools

import jax
import jax.numpy as jnp
from jax import lax
from jax.experimental import pallas as pl
from jax.experimental.pallas import tpu as pltpu
from jax.experimental.pallas import tpu_sc as plsc

D = 64
SCALE = 8.0
B_TOTAL = 4096 * 200
NC = 2
NS = 16
NW = NC * NS
B_PER_W = B_TOTAL // NW
CHUNK = 128
NBUF = 4
N_CHUNKS = B_PER_W // CHUNK

assert B_PER_W * NW == B_TOTAL
assert N_CHUNKS * CHUNK == B_PER_W
assert (N_CHUNKS - 2 * NBUF) % NBUF == 0


def _scale_chunk(rin, rout, b):
    def row(r, carry):
        for j in range(D // 16):
            s = pl.ds(j * 16, 16)
            rout[b, r, s] = rin[b, r, s] * SCALE
        return carry
    lax.fori_loop(0, CHUNK, row, 0)


def _make_emb():
    mesh = plsc.VectorSubcoreMesh(core_axis_name="c", subcore_axis_name="s")

    @functools.partial(
        pl.kernel,
        mesh=mesh,
        out_type=jax.ShapeDtypeStruct((B_TOTAL, D), jnp.float32),
        compiler_params=pltpu.CompilerParams(use_tc_tiling_on_sc=False),
        scratch_types=(
            [pltpu.VMEM((B_PER_W,), jnp.int32),
             pltpu.VMEM((NBUF, CHUNK, D), jnp.float32),
             pltpu.VMEM((NBUF, CHUNK, D), jnp.float32)]
            + [pltpu.SemaphoreType.DMA] * (1 + 2 * NBUF)
        ),
    )
    def emb(x_hbm, table_hbm, out_hbm, idx_v, rin_v, rout_v, *sems):
        isem = sems[0]
        gsem = sems[1:1 + NBUF]
        ssem = sems[1 + NBUF:]
        wid = lax.axis_index("s") * NC + lax.axis_index("c")
        base = wid * B_PER_W

        pltpu.async_copy(x_hbm.at[pl.ds(base, B_PER_W)], idx_v, isem)
        pltpu.make_async_copy(
            x_hbm.at[pl.ds(base, B_PER_W)], idx_v, isem).wait()

        def issue_gather(g, b):
            pltpu.async_copy(
                table_hbm.at[idx_v.at[pl.ds(g * CHUNK, CHUNK)]],
                rin_v.at[b], gsem[b])

        def wait_gather(b):
            pltpu.make_async_copy(
                table_hbm.at[idx_v.at[pl.ds(0, CHUNK)]], rin_v.at[b],
                gsem[b]).wait()

        def issue_scatter(g, b):
            off = base + g * CHUNK
            pltpu.async_copy(
                rout_v.at[b], out_hbm.at[pl.ds(off, CHUNK)], ssem[b])

        def wait_scatter(g, b):
            off = base + g * CHUNK
            pltpu.make_async_copy(
                rout_v.at[b], out_hbm.at[pl.ds(off, CHUNK)], ssem[b]).wait()

        for b in range(NBUF):
            issue_gather(b, b)

        for b in range(NBUF):
            wait_gather(b)
            _scale_chunk(rin_v, rout_v, b)
            issue_scatter(b, b)
            issue_gather(b + NBUF, b)

        def outer(i, carry):
            g0 = NBUF + i * NBUF
            for b in range(NBUF):
                g = g0 + b
                wait_gather(b)
                wait_scatter(g, b)
                _scale_chunk(rin_v, rout_v, b)
                issue_scatter(g, b)
                issue_gather(g + NBUF, b)
            return carry
        lax.fori_loop(0, (N_CHUNKS - 2 * NBUF) // NBUF, outer, 0)

        for b in range(NBUF):
            g = N_CHUNKS - NBUF + b
            wait_gather(b)
            wait_scatter(g, b)
            _scale_chunk(rin_v, rout_v, b)
            issue_scatter(g, b)

        for b in range(NBUF):
            wait_scatter(N_CHUNKS - NBUF + b, b)

    return emb


_emb = _make_emb()


def kernel(x, table):
    out = _emb(x.reshape(B_TOTAL).astype(jnp.int32), table)
    return out.reshape(4096, 200, D)
